# baseline (speedup 1.0000x reference)
import functools

import jax
import jax.numpy as jnp
from jax import lax
from jax.experimental import pallas as pl
from jax.experimental.pallas import tpu as pltpu
from jax.experimental.pallas import tpu_sc as plsc

N = 10000
NPAD = 10240
E = 320000
D = 128

NC = 2
NS = 16
NW = NC * NS
CHUNK = 128
NCHUNKS = 80
EPAD = NW * NCHUNKS * CHUNK
RPS = NPAD // NS
NBUF = 4

_mesh = plsc.VectorSubcoreMesh(core_axis_name="c", subcore_axis_name="s")

import dataclasses
_cp = pltpu.CompilerParams()
if "needs_layout_passes" in pltpu.CompilerParams.__dataclass_fields__:
    _cp = dataclasses.replace(_cp, needs_layout_passes=False)


@functools.partial(
    pl.kernel,
    out_type=jax.ShapeDtypeStruct((NC, NPAD, 16), jnp.float32),
    mesh=_mesh,
    scratch_types=[
        pltpu.VMEM((NCHUNKS, CHUNK), jnp.int32),
        pltpu.VMEM((NPAD,), jnp.float32),
        pltpu.VMEM((RPS,), jnp.float32),
        pltpu.VMEM((RPS,), jnp.float32),
        pltpu.VMEM((RPS, 16), jnp.float32),
        pltpu.VMEM_SHARED((NS, NPAD), jnp.float32),
    ],
    compiler_params=_cp,
)
def _deg_kernel(dstp_hbm, out_hbm, didx, hist, accv, tbuf, loc, stage):
    core = lax.axis_index("c")
    sub = lax.axis_index("s")
    wid = sub * NC + core

    pltpu.sync_copy(dstp_hbm.at[pl.ds(wid * NCHUNKS, NCHUNKS)], didx)

    zero = jnp.zeros((16,), jnp.float32)

    @pl.loop(0, NPAD // 16)
    def _(i):
        hist[pl.ds(i * 16, 16)] = zero

    ones16 = jnp.full((16,), 1.0, jnp.float32)

    @pl.loop(0, NCHUNKS)
    def _(j):
        for c in range(CHUNK // 16):
            idx = didx[j, pl.ds(c * 16, 16)]
            plsc.addupdate_scatter(hist, [idx], ones16)

    pltpu.sync_copy(hist, stage.at[sub])
    plsc.subcore_barrier()

    @pl.loop(0, RPS // 16)
    def _(i):
        accv[pl.ds(i * 16, 16)] = zero

    for t in range(NS):
        pltpu.sync_copy(stage.at[t, pl.ds(sub * RPS, RPS)], tbuf)

        @pl.loop(0, RPS // 16)
        def _(i):
            accv[pl.ds(i * 16, 16)] = accv[pl.ds(i * 16, 16)] + tbuf[pl.ds(i * 16, 16)]

    @pl.loop(0, RPS // 16)
    def _(g):
        v = accv[pl.ds(g * 16, 16)]
        for r in range(16):
            loc[g * 16 + r, pl.ds(0, 16)] = jnp.broadcast_to(v[r], (16,))

    pltpu.sync_copy(loc, out_hbm.at[core, pl.ds(sub * RPS, RPS)])


def _make_agg():
  @functools.partial(
      pl.kernel,
      out_type=jax.ShapeDtypeStruct((NC, NPAD, D), jnp.float32),
      mesh=_mesh,
      scratch_types=[
          pltpu.VMEM((NCHUNKS, CHUNK), jnp.int32),
          pltpu.VMEM((2, CHUNK), jnp.int32),
          pltpu.VMEM((2, CHUNK, D), jnp.float32),
          pltpu.VMEM_SHARED((NPAD, D), jnp.float32),
          pltpu.SemaphoreType.DMA,
          pltpu.SemaphoreType.DMA,
          pltpu.SemaphoreType.DMA,
          pltpu.SemaphoreType.DMA,
      ],
  )
  def _agg_kernel(srcp_hbm, dstp_hbm, xs_hbm, out_hbm, sidx, didx, rows, acc,
                  g0, g1, d0, d1):
      core = lax.axis_index("c")
      sub = lax.axis_index("s")
      wid = sub * NC + core
      gsem = [g0, g1]
      dsem = [d0, d1]

      pltpu.sync_copy(srcp_hbm.at[pl.ds(wid * NCHUNKS, NCHUNKS)], sidx)

      zero = jnp.zeros((16,), jnp.float32)

      @pl.loop(0, CHUNK)
      def _(r):
          for c in range(D // 16):
              rows[0, r, pl.ds(c * 16, 16)] = zero

      for k in range(RPS // CHUNK):
          pltpu.sync_copy(rows.at[0], acc.at[pl.ds(sub * RPS + k * CHUNK, CHUNK)])
      plsc.subcore_barrier()

      for b in range(2):
          pltpu.async_copy(xs_hbm.at[sidx.at[b]], rows.at[b], gsem[b])
          pltpu.async_copy(dstp_hbm.at[wid * NCHUNKS + b], didx.at[b], dsem[b])

      @pl.loop(0, NCHUNKS - 2, step=2)
      def _(j):
          for b in range(2):
              pltpu.make_async_copy(xs_hbm.at[sidx.at[b]], rows.at[b], gsem[b]).wait()
              pltpu.make_async_copy(dstp_hbm.at[0], didx.at[b], dsem[b]).wait()
              pltpu.sync_copy(rows.at[b], acc.at[didx.at[b]], add=True)
              pltpu.async_copy(xs_hbm.at[sidx.at[j + 2 + b]], rows.at[b], gsem[b])
              pltpu.async_copy(dstp_hbm.at[wid * NCHUNKS + j + 2 + b], didx.at[b],
                               dsem[b])

      for b in range(2):
          pltpu.make_async_copy(xs_hbm.at[sidx.at[b]], rows.at[b], gsem[b]).wait()
          pltpu.make_async_copy(dstp_hbm.at[0], didx.at[b], dsem[b]).wait()
          pltpu.sync_copy(rows.at[b], acc.at[didx.at[b]], add=True)

      plsc.subcore_barrier()
      for k in range(RPS // CHUNK):
          r0 = sub * RPS + k * CHUNK
          pltpu.sync_copy(acc.at[pl.ds(r0, CHUNK)], out_hbm.at[core, pl.ds(r0, CHUNK)])

  return _agg_kernel


_agg1 = _make_agg()
_agg2 = _make_agg()


BLK = 1024
DEGW = 16


def _dinv_of(degp_ref):
    deg = degp_ref[0, :, 0:1] + degp_ref[1, :, 0:1] + 1.0
    return lax.rsqrt(deg)


def _tc1_body(x_ref, w_ref, degp_ref, xs_ref):
    dinv = _dinv_of(degp_ref)
    xw = jnp.dot(x_ref[...], w_ref[...], preferred_element_type=jnp.float32)
    xs_ref[...] = xw * dinv


def _tc2_body(p_ref, xs1_ref, degp_ref, b_ref, w_ref, xs2_ref):
    dinv = _dinv_of(degp_ref)
    s = p_ref[0] + p_ref[1] + xs1_ref[...]
    h = jnp.maximum(dinv * s + b_ref[...], 0.0)
    hw = jnp.dot(h, w_ref[...], preferred_element_type=jnp.float32)
    xs2_ref[...] = hw * dinv


def _tc3_body(p_ref, xs2_ref, degp_ref, b_ref, out_ref):
    dinv = _dinv_of(degp_ref)
    s = p_ref[0] + p_ref[1] + xs2_ref[...]
    out_ref[...] = dinv * s + b_ref[...]


_row_spec = pl.BlockSpec((BLK, D), lambda i: (i, 0))
_p_spec = pl.BlockSpec((NC, BLK, D), lambda i: (0, i, 0))
_degp_spec = pl.BlockSpec((NC, BLK, DEGW), lambda i: (0, i, 0))
_w_spec = pl.BlockSpec((D, D), lambda i: (0, 0))
_b_spec = pl.BlockSpec((1, D), lambda i: (0, 0))

_tc1 = pl.pallas_call(
    _tc1_body,
    grid=(NPAD // BLK,),
    in_specs=[_row_spec, _w_spec, _degp_spec],
    out_specs=_row_spec,
    out_shape=jax.ShapeDtypeStruct((NPAD, D), jnp.float32),
)

_tc2 = pl.pallas_call(
    _tc2_body,
    grid=(NPAD // BLK,),
    in_specs=[_p_spec, _row_spec, _degp_spec, _b_spec, _w_spec],
    out_specs=_row_spec,
    out_shape=jax.ShapeDtypeStruct((NPAD, D), jnp.float32),
)

_tc3 = pl.pallas_call(
    _tc3_body,
    grid=(NPAD // BLK,),
    in_specs=[_p_spec, _row_spec, _degp_spec, _b_spec],
    out_specs=_row_spec,
    out_shape=jax.ShapeDtypeStruct((NPAD, D), jnp.float32),
)


def kernel(x, edge_index, W1, b1, W2, b2):
    src = edge_index[0].astype(jnp.int32)
    dst = edge_index[1].astype(jnp.int32)
    pad = EPAD - E
    srcp = jnp.concatenate([src, jnp.zeros((pad,), jnp.int32)])
    srcp = srcp.reshape(NW * NCHUNKS, CHUNK)
    dstp = jnp.concatenate([dst, jnp.full((pad,), NPAD - 1, jnp.int32)])
    dstp = dstp.reshape(NW * NCHUNKS, CHUNK)
    b1r = b1.reshape(1, D)
    b2r = b2.reshape(1, D)
    xp = jnp.pad(x, ((0, NPAD - N), (0, 0)))

    degp = _deg_kernel(dstp)
    xs1 = _tc1(xp, W1, degp)
    p1 = _agg1(srcp, dstp, xs1)
    xs2 = _tc2(p1, xs1, degp, b1r, W2)
    p2 = _agg2(srcp, dstp, xs2)
    return _tc3(p2, xs2, degp, b2r)[:N]
